# speedup vs baseline: 2.9824x; 2.9824x over previous
import functools

import jax
import jax.numpy as jnp
from jax import lax
from jax.experimental import pallas as pl
from jax.experimental.pallas import tpu as pltpu
from jax.experimental.pallas import tpu_sc as plsc

_VOCAB = 1000
_D = 128
_POS = 4
_NC = 2
_NS = 16
_NW = _NC * _NS
_LANES = 16


def _table_body(embed_ref, wt_ref, b_ref, tbl_ref):
    bias = b_ref[...] * (1.0 / _POS)
    for j in range(_POS):
        tbl_ref[j] = (
            lax.dot_general(
                embed_ref[...],
                wt_ref[j],
                dimension_numbers=(((1,), (1,)), ((), ())),
                preferred_element_type=jnp.float32,
            )
            + bias[None, :]
        )


def _make_fused_table(embed, W, b):
    wt = W.reshape(_D, _POS, _D).transpose(1, 0, 2)
    tbl = pl.pallas_call(
        _table_body,
        out_shape=jax.ShapeDtypeStruct((_POS, _VOCAB, _D), jnp.float32),
    )(embed, wt, b)
    return tbl.reshape(_POS * _VOCAB, _D)


def _sc_body(tok_ref, tbl_ref, out_ref, idx_v, rows_v, outc_v, sem,
             *, b_per_w, cb):
    wid = lax.axis_index("s") * _NC + lax.axis_index("c")
    base = wid * b_per_w
    n_chunks = b_per_w // cb
    g = cb * _POS

    offs = lax.rem(lax.iota(jnp.int32, _LANES), _POS) * _VOCAB

    def chunk_body(c, _):
        row0 = base + c * cb
        pltpu.sync_copy(tok_ref.at[pl.ds(row0 * _POS, g)], idx_v)
        for s in range(g // _LANES):
            sl = pl.ds(s * _LANES, _LANES)
            idx_v[sl] = idx_v[sl] + offs
        pltpu.async_copy(tbl_ref.at[idx_v], rows_v, sem).wait()

        def row_body(r, _):
            for col in range(_D // _LANES):
                sl = pl.ds(col * _LANES, _LANES)
                acc = rows_v[_POS * r, sl]
                for j in range(1, _POS):
                    acc = acc + rows_v[_POS * r + j, sl]
                outc_v[r, sl] = acc
            return 0

        lax.fori_loop(0, cb, row_body, 0)
        pltpu.sync_copy(outc_v, out_ref.at[pl.ds(row0, cb)])
        return 0

    lax.fori_loop(0, n_chunks, chunk_body, 0)


def _gather_sum(tokens_flat, tbl, batch):
    b_per_w = batch // _NW
    cb = 32
    mesh = plsc.VectorSubcoreMesh(
        core_axis_name="c", subcore_axis_name="s",
        num_cores=_NC, num_subcores=_NS,
    )
    g = cb * _POS
    run = pl.kernel(
        functools.partial(_sc_body, b_per_w=b_per_w, cb=cb),
        out_type=jax.ShapeDtypeStruct((batch, _D), jnp.float32),
        mesh=mesh,
        scratch_types=[
            pltpu.VMEM((g,), jnp.int32),
            pltpu.VMEM((g, _D), jnp.float32),
            pltpu.VMEM((cb, _D), jnp.float32),
            pltpu.SemaphoreType.DMA,
        ],
    )
    return run(tokens_flat, tbl)


def kernel(ctrl_tokens, embed, W, b):
    batch = ctrl_tokens.shape[0]
    tokens_flat = ctrl_tokens.astype(jnp.int32).reshape(-1)
    tbl = _make_fused_table(embed, W, b)
    out = _gather_sum(tokens_flat, tbl, batch)
    return out[..., None]


# speedup vs baseline: 5.1091x; 1.7131x over previous
import functools

import jax
import jax.numpy as jnp
from jax import lax
from jax.experimental import pallas as pl
from jax.experimental.pallas import tpu as pltpu
from jax.experimental.pallas import tpu_sc as plsc

_VOCAB = 1000
_D = 128
_POS = 4
_NC = 2
_NS = 16
_NW = _NC * _NS
_LANES = 16


def _table_body(embed_ref, wt_ref, b_ref, tbl_ref):
    bias = b_ref[...] * (1.0 / _POS)
    for j in range(_POS):
        tbl_ref[j] = (
            lax.dot_general(
                embed_ref[...],
                wt_ref[j],
                dimension_numbers=(((1,), (1,)), ((), ())),
                preferred_element_type=jnp.float32,
            )
            + bias[None, :]
        )


def _make_fused_table(embed, W, b):
    wt = W.reshape(_D, _POS, _D).transpose(1, 0, 2)
    tbl = pl.pallas_call(
        _table_body,
        out_shape=jax.ShapeDtypeStruct((_POS, _VOCAB, _D), jnp.float32),
    )(embed, wt, b)
    return tbl.reshape(_POS * _VOCAB, _D)


def _sc_body(tok_ref, tbl_ref, out_ref,
             idx_v, rows0, rows1, outc0, outc1,
             sem_g0, sem_g1, sem_o0, sem_o1,
             *, b_per_w, cb):
    wid = lax.axis_index("s") * _NC + lax.axis_index("c")
    base = wid * b_per_w
    n_chunks = b_per_w // cb
    g = cb * _POS
    rows_bufs = (rows0, rows1)
    outc_bufs = (outc0, outc1)
    sem_g = (sem_g0, sem_g1)
    sem_o = (sem_o0, sem_o1)

    pltpu.sync_copy(tok_ref.at[wid], idx_v)

    offs = lax.rem(lax.iota(jnp.int32, _LANES), _POS) * _VOCAB

    @plsc.parallel_loop(0, n_chunks, 1, unroll=2)
    def _(c):
        for s in range(g // _LANES):
            sl = pl.ds(s * _LANES, _LANES)
            idx_v[c, sl] = idx_v[c, sl] + offs

    pltpu.async_copy(tbl_ref.at[idx_v.at[0]], rows0, sem_g0)
    pltpu.async_copy(tbl_ref.at[idx_v.at[1]], rows1, sem_g1)

    def pair_body(p, _):
        for bsel in range(2):
            c = p * 2 + bsel
            rows_b = rows_bufs[bsel]
            outc_b = outc_bufs[bsel]
            pltpu.make_async_copy(tbl_ref.at[idx_v.at[0]], rows_b,
                                  sem_g[bsel]).wait()
            @pl.when(c >= 2)
            def _():
                pltpu.make_async_copy(outc_b, out_ref.at[pl.ds(0, cb)],
                                      sem_o[bsel]).wait()

            @plsc.parallel_loop(0, cb, 1, unroll=2)
            def _(r):
                for col in range(_D // _LANES):
                    sl = pl.ds(col * _LANES, _LANES)
                    acc = rows_b[_POS * r, sl]
                    for j in range(1, _POS):
                        acc = acc + rows_b[_POS * r + j, sl]
                    outc_b[r, sl] = acc

            row0 = base + c * cb
            pltpu.async_copy(outc_b, out_ref.at[pl.ds(row0, cb)], sem_o[bsel])

            @pl.when(c + 2 < n_chunks)
            def _():
                pltpu.async_copy(tbl_ref.at[idx_v.at[c + 2]], rows_b,
                                 sem_g[bsel])
        return 0

    lax.fori_loop(0, n_chunks // 2, pair_body, 0)

    for bsel in range(2):
        pltpu.make_async_copy(outc_bufs[bsel], out_ref.at[pl.ds(0, cb)],
                              sem_o[bsel]).wait()


def _gather_sum(tokens3d, tbl, batch):
    b_per_w = batch // _NW
    cb = 32
    n_chunks = b_per_w // cb
    mesh = plsc.VectorSubcoreMesh(
        core_axis_name="c", subcore_axis_name="s",
        num_cores=_NC, num_subcores=_NS,
    )
    g = cb * _POS
    run = pl.kernel(
        functools.partial(_sc_body, b_per_w=b_per_w, cb=cb),
        out_type=jax.ShapeDtypeStruct((batch, _D), jnp.float32),
        mesh=mesh,
        scratch_types=[
            pltpu.VMEM((n_chunks, g), jnp.int32),
            pltpu.VMEM((g, _D), jnp.float32),
            pltpu.VMEM((g, _D), jnp.float32),
            pltpu.VMEM((cb, _D), jnp.float32),
            pltpu.VMEM((cb, _D), jnp.float32),
            pltpu.SemaphoreType.DMA,
            pltpu.SemaphoreType.DMA,
            pltpu.SemaphoreType.DMA,
            pltpu.SemaphoreType.DMA,
        ],
    )
    return run(tokens3d, tbl)


def kernel(ctrl_tokens, embed, W, b):
    batch = ctrl_tokens.shape[0]
    b_per_w = batch // _NW
    cb = 32
    tokens3d = ctrl_tokens.astype(jnp.int32).reshape(
        _NW, b_per_w // cb, cb * _POS)
    tbl = _make_fused_table(embed, W, b)
    out = _gather_sum(tokens3d, tbl, batch)
    return out[..., None]


# speedup vs baseline: 5.2654x; 1.0306x over previous
import functools

import jax
import jax.numpy as jnp
from jax import lax
from jax.experimental import pallas as pl
from jax.experimental.pallas import tpu as pltpu
from jax.experimental.pallas import tpu_sc as plsc

_VOCAB = 1000
_D = 128
_POS = 4
_NC = 2
_NS = 16
_NW = _NC * _NS
_LANES = 16


def _table_body(embed_ref, w_ref, b_ref, tbl_ref):
    bias = b_ref[...] * (1.0 / _POS)
    for j in range(_POS):
        w_j = w_ref[:, j * _D:(j + 1) * _D]
        tbl_ref[j] = (
            lax.dot_general(
                embed_ref[...],
                w_j,
                dimension_numbers=(((1,), (1,)), ((), ())),
                preferred_element_type=jnp.float32,
            )
            + bias[None, :]
        )


def _make_fused_table(embed, W, b):
    tbl = pl.pallas_call(
        _table_body,
        out_shape=jax.ShapeDtypeStruct((_POS, _VOCAB, _D), jnp.float32),
    )(embed, W, b)
    return tbl.reshape(_POS * _VOCAB, _D)


def _sc_body(tok_ref, tbl_ref, out_ref,
             idx_v, rows0, rows1, outc0, outc1,
             sem_g0, sem_g1, sem_o0, sem_o1,
             *, b_per_w, cb):
    wid = lax.axis_index("s") * _NC + lax.axis_index("c")
    base = wid * b_per_w
    n_chunks = b_per_w // cb
    g = cb * _POS
    rows_bufs = (rows0, rows1)
    outc_bufs = (outc0, outc1)
    sem_g = (sem_g0, sem_g1)
    sem_o = (sem_o0, sem_o1)

    pltpu.sync_copy(tok_ref.at[wid], idx_v)

    offs = lax.rem(lax.iota(jnp.int32, _LANES), _POS) * _VOCAB

    @plsc.parallel_loop(0, n_chunks, 1, unroll=2)
    def _(c):
        for s in range(g // _LANES):
            sl = pl.ds(s * _LANES, _LANES)
            idx_v[c, sl] = idx_v[c, sl] + offs

    pltpu.async_copy(tbl_ref.at[idx_v.at[0]], rows0, sem_g0)
    pltpu.async_copy(tbl_ref.at[idx_v.at[1]], rows1, sem_g1)

    def pair_body(p, _):
        for bsel in range(2):
            c = p * 2 + bsel
            rows_b = rows_bufs[bsel]
            outc_b = outc_bufs[bsel]
            pltpu.make_async_copy(tbl_ref.at[idx_v.at[0]], rows_b,
                                  sem_g[bsel]).wait()
            @pl.when(c >= 2)
            def _():
                pltpu.make_async_copy(outc_b, out_ref.at[pl.ds(0, cb * _D)],
                                      sem_o[bsel]).wait()

            @plsc.parallel_loop(0, cb, 1, unroll=2)
            def _(r):
                for col in range(_D // _LANES):
                    sl = pl.ds(col * _LANES, _LANES)
                    acc = rows_b[_POS * r, sl]
                    for j in range(1, _POS):
                        acc = acc + rows_b[_POS * r + j, sl]
                    outc_b[pl.ds(r * _D + col * _LANES, _LANES)] = acc

            row0 = base + c * cb
            pltpu.async_copy(outc_b, out_ref.at[pl.ds(row0 * _D, cb * _D)],
                             sem_o[bsel])

            @pl.when(c + 2 < n_chunks)
            def _():
                pltpu.async_copy(tbl_ref.at[idx_v.at[c + 2]], rows_b,
                                 sem_g[bsel])
        return 0

    lax.fori_loop(0, n_chunks // 2, pair_body, 0)

    for bsel in range(2):
        pltpu.make_async_copy(outc_bufs[bsel], out_ref.at[pl.ds(0, cb * _D)],
                              sem_o[bsel]).wait()


def _gather_sum(tokens3d, tbl, batch):
    b_per_w = batch // _NW
    cb = 32
    n_chunks = b_per_w // cb
    mesh = plsc.VectorSubcoreMesh(
        core_axis_name="c", subcore_axis_name="s",
        num_cores=_NC, num_subcores=_NS,
    )
    g = cb * _POS
    run = pl.kernel(
        functools.partial(_sc_body, b_per_w=b_per_w, cb=cb),
        out_type=jax.ShapeDtypeStruct((batch * _D,), jnp.float32),
        mesh=mesh,
        scratch_types=[
            pltpu.VMEM((n_chunks, g), jnp.int32),
            pltpu.VMEM((g, _D), jnp.float32),
            pltpu.VMEM((g, _D), jnp.float32),
            pltpu.VMEM((cb * _D,), jnp.float32),
            pltpu.VMEM((cb * _D,), jnp.float32),
            pltpu.SemaphoreType.DMA,
            pltpu.SemaphoreType.DMA,
            pltpu.SemaphoreType.DMA,
            pltpu.SemaphoreType.DMA,
        ],
    )
    return run(tokens3d, tbl)


def kernel(ctrl_tokens, embed, W, b):
    batch = ctrl_tokens.shape[0]
    b_per_w = batch // _NW
    cb = 32
    tokens3d = ctrl_tokens.astype(jnp.int32).reshape(
        _NW, b_per_w // cb, cb * _POS)
    tbl = _make_fused_table(embed, W, b)
    out = _gather_sum(tokens3d, tbl, batch)
    return out.reshape(batch, _D, 1)


# speedup vs baseline: 6.5480x; 1.2436x over previous
import functools

import jax
import jax.numpy as jnp
from jax import lax
from jax.experimental import pallas as pl
from jax.experimental.pallas import tpu as pltpu
from jax.experimental.pallas import tpu_sc as plsc

_VOCAB = 1000
_D = 128
_POS = 4
_NC = 2
_NS = 16
_NW = _NC * _NS
_LANES = 16


def _table_body(embed_ref, w_ref, b_ref, tbl_ref):
    bias = b_ref[...] * (1.0 / _POS)
    for j in range(_POS):
        w_j = w_ref[:, j * _D:(j + 1) * _D]
        tbl_ref[j] = (
            lax.dot_general(
                embed_ref[...],
                w_j,
                dimension_numbers=(((1,), (1,)), ((), ())),
                preferred_element_type=jnp.float32,
            )
            + bias[None, :]
        )


def _make_fused_table(embed, W, b):
    tbl = pl.pallas_call(
        _table_body,
        out_shape=jax.ShapeDtypeStruct((_POS, _VOCAB, _D), jnp.float32),
    )(embed, W, b)
    return tbl.reshape(_POS * _VOCAB, _D)


def _sc_body(tok_ref, tbl_ref, out_ref,
             tok_v, idx_v, rows0, rows1, outc0, outc1,
             sem_g0, sem_g1, sem_o0, sem_o1,
             *, batch, b_per_w, cb):
    wid = lax.axis_index("s") * _NC + lax.axis_index("c")
    base = wid * b_per_w
    n_chunks = b_per_w // cb
    g = cb * _POS
    rows_bufs = (rows0, rows1)
    outc_bufs = (outc0, outc1)
    sem_g = (sem_g0, sem_g1)
    sem_o = (sem_o0, sem_o1)

    for j in range(_POS):
        pltpu.sync_copy(tok_ref.at[pl.ds(j * batch + base, b_per_w)],
                        tok_v.at[pl.ds(j * b_per_w, b_per_w)])

    @plsc.parallel_loop(0, n_chunks, 1, unroll=2)
    def _(c):
        for j in range(_POS):
            for s in range(cb // _LANES):
                src = pl.ds(j * b_per_w + c * cb + s * _LANES, _LANES)
                dst = pl.ds(j * cb + s * _LANES, _LANES)
                idx_v[c, dst] = tok_v[src] + (j * _VOCAB)

    pltpu.async_copy(tbl_ref.at[idx_v.at[0]], rows0, sem_g0)
    pltpu.async_copy(tbl_ref.at[idx_v.at[1]], rows1, sem_g1)

    def pair_body(p, _):
        for bsel in range(2):
            c = p * 2 + bsel
            rows_b = rows_bufs[bsel]
            outc_b = outc_bufs[bsel]
            pltpu.make_async_copy(tbl_ref.at[idx_v.at[0]], rows_b,
                                  sem_g[bsel]).wait()
            @pl.when(c >= 2)
            def _():
                pltpu.make_async_copy(outc_b, out_ref.at[pl.ds(0, cb * _D)],
                                      sem_o[bsel]).wait()

            @plsc.parallel_loop(0, cb, 1, unroll=2)
            def _(r):
                for col in range(_D // _LANES):
                    sl = pl.ds(col * _LANES, _LANES)
                    acc = rows_b[r, sl]
                    for j in range(1, _POS):
                        acc = acc + rows_b[j * cb + r, sl]
                    outc_b[pl.ds(r * _D + col * _LANES, _LANES)] = acc

            row0 = base + c * cb
            pltpu.async_copy(outc_b, out_ref.at[pl.ds(row0 * _D, cb * _D)],
                             sem_o[bsel])

            @pl.when(c + 2 < n_chunks)
            def _():
                pltpu.async_copy(tbl_ref.at[idx_v.at[c + 2]], rows_b,
                                 sem_g[bsel])
        return 0

    lax.fori_loop(0, n_chunks // 2, pair_body, 0)

    for bsel in range(2):
        pltpu.make_async_copy(outc_bufs[bsel], out_ref.at[pl.ds(0, cb * _D)],
                              sem_o[bsel]).wait()


def _gather_sum(tokens_pm, tbl, batch):
    b_per_w = batch // _NW
    cb = 32
    n_chunks = b_per_w // cb
    mesh = plsc.VectorSubcoreMesh(
        core_axis_name="c", subcore_axis_name="s",
        num_cores=_NC, num_subcores=_NS,
    )
    g = cb * _POS
    run = pl.kernel(
        functools.partial(_sc_body, batch=batch, b_per_w=b_per_w, cb=cb),
        out_type=jax.ShapeDtypeStruct((batch * _D,), jnp.float32),
        mesh=mesh,
        scratch_types=[
            pltpu.VMEM((_POS * b_per_w,), jnp.int32),
            pltpu.VMEM((n_chunks, g), jnp.int32),
            pltpu.VMEM((g, _D), jnp.float32),
            pltpu.VMEM((g, _D), jnp.float32),
            pltpu.VMEM((cb * _D,), jnp.float32),
            pltpu.VMEM((cb * _D,), jnp.float32),
            pltpu.SemaphoreType.DMA,
            pltpu.SemaphoreType.DMA,
            pltpu.SemaphoreType.DMA,
            pltpu.SemaphoreType.DMA,
        ],
    )
    return run(tokens_pm, tbl)


def kernel(ctrl_tokens, embed, W, b):
    batch = ctrl_tokens.shape[0]
    tokens_pm = ctrl_tokens.astype(jnp.int32).T.reshape(-1)
    tbl = _make_fused_table(embed, W, b)
    out = _gather_sum(tokens_pm, tbl, batch)
    return out.reshape(batch, _D, 1)
